# Initial kernel scaffold; baseline (speedup 1.0000x reference)
import functools

import jax
import jax.numpy as jnp
from jax.experimental import pallas as pl

_NE = 16
_MIN_EXPERTS = 2


def _gating_body(x_ref, w_ref, g_ref, rw_ref, lg_ref, am_ref):
    x = x_ref[...]
    w = w_ref[...]
    g = g_ref[...]

    wn = w * jax.lax.rsqrt(jnp.maximum(jnp.sum(w * w, axis=0, keepdims=True),
                                       1e-24))
    xw = jnp.dot(x, wn, preferred_element_type=jnp.float32)
    rn = jnp.sqrt(jnp.sum(x * x, axis=1, keepdims=True))
    inv_rn = 1.0 / jnp.maximum(rn, 1e-12)
    logits = xw * inv_rn - jax.nn.sigmoid(g)

    gated = jnp.maximum(logits, 0.0)
    act = (logits > 0.0).astype(jnp.float32)
    inactive = jnp.sum(act, axis=1, keepdims=True) == 0.0

    col = jax.lax.broadcasted_iota(jnp.int32, logits.shape, 1)
    a1 = jnp.argmax(logits, axis=1, keepdims=True)
    masked1 = jnp.where(col == a1, -jnp.inf, logits)
    a2 = jnp.argmax(masked1, axis=1, keepdims=True)
    fb = ((col == a1) | (col == a2)).astype(jnp.float32)

    mask = jnp.where(inactive, fb, act)
    neg_inf = jnp.float32(-jnp.inf)
    ml = jnp.where(mask > 0.0, gated, neg_inf)
    mx = jnp.max(ml, axis=1, keepdims=True)
    e = jnp.exp(ml - mx)
    rw = e / jnp.sum(e, axis=1, keepdims=True)

    rw_ref[...] = rw
    lg_ref[...] = logits
    am_ref[...] = mask


@functools.partial(jax.jit, static_argnames=())
def kernel(hidden_states, sim_matrix, gates):
    b, t, c = hidden_states.shape
    n = b * t
    flat = hidden_states.reshape(n, c)
    g2 = gates.reshape(1, _NE)

    tile_m = 1024
    grid = (n // tile_m,)

    out_shape = [
        jax.ShapeDtypeStruct((n, _NE), jnp.float32),
        jax.ShapeDtypeStruct((n, _NE), jnp.float32),
        jax.ShapeDtypeStruct((n, _NE), jnp.float32),
    ]
    out_spec = pl.BlockSpec((tile_m, _NE), lambda i: (i, 0))

    rw, lg, am = pl.pallas_call(
        _gating_body,
        grid=grid,
        in_specs=[
            pl.BlockSpec((tile_m, c), lambda i: (i, 0)),
            pl.BlockSpec((c, _NE), lambda i: (0, 0)),
            pl.BlockSpec((1, _NE), lambda i: (0, 0)),
        ],
        out_specs=[out_spec, out_spec, out_spec],
        out_shape=out_shape,
    )(flat, sim_matrix, g2)
    return rw, lg, am


# speedup vs baseline: 2.9731x; 2.9731x over previous
import functools

import jax
import jax.numpy as jnp
from jax.experimental import pallas as pl

_NE = 16
_MIN_EXPERTS = 2


def _gating_body(x_ref, w_ref, g_ref, rw_ref, lg_ref, am_ref):
    x = x_ref[...]
    w = w_ref[...]
    g = g_ref[...]

    wn = w * jax.lax.rsqrt(jnp.maximum(jnp.sum(w * w, axis=0, keepdims=True),
                                       1e-24))
    rn = jnp.sqrt(jnp.sum(x * x, axis=1, keepdims=True))
    xn = x / jnp.maximum(rn, 1e-12)
    xw = jnp.dot(xn.astype(jnp.bfloat16), wn.astype(jnp.bfloat16),
                 preferred_element_type=jnp.float32)
    logits = xw - jax.nn.sigmoid(g)

    gated = jnp.maximum(logits, 0.0)
    act = (logits > 0.0).astype(jnp.float32)
    inactive = jnp.sum(act, axis=1, keepdims=True) == 0.0

    col = jax.lax.broadcasted_iota(jnp.int32, logits.shape, 1)
    a1 = jnp.argmax(logits, axis=1, keepdims=True)
    masked1 = jnp.where(col == a1, -jnp.inf, logits)
    a2 = jnp.argmax(masked1, axis=1, keepdims=True)
    fb = ((col == a1) | (col == a2)).astype(jnp.float32)

    mask = jnp.where(inactive, fb, act)
    neg_inf = jnp.float32(-jnp.inf)
    ml = jnp.where(mask > 0.0, gated, neg_inf)
    mx = jnp.max(ml, axis=1, keepdims=True)
    e = jnp.exp(ml - mx)
    rw = e / jnp.sum(e, axis=1, keepdims=True)

    rw_ref[...] = rw
    lg_ref[...] = logits
    am_ref[...] = mask


@functools.partial(jax.jit, static_argnames=())
def kernel(hidden_states, sim_matrix, gates):
    b, t, c = hidden_states.shape
    n = b * t
    flat = hidden_states.reshape(n, c)
    g2 = gates.reshape(1, _NE)

    tile_m = 1024
    grid = (n // tile_m,)

    out_shape = [
        jax.ShapeDtypeStruct((n, _NE), jnp.float32),
        jax.ShapeDtypeStruct((n, _NE), jnp.float32),
        jax.ShapeDtypeStruct((n, _NE), jnp.float32),
    ]
    out_spec = pl.BlockSpec((tile_m, _NE), lambda i: (i, 0))

    rw, lg, am = pl.pallas_call(
        _gating_body,
        grid=grid,
        in_specs=[
            pl.BlockSpec((tile_m, c), lambda i: (i, 0)),
            pl.BlockSpec((c, _NE), lambda i: (0, 0)),
            pl.BlockSpec((1, _NE), lambda i: (0, 0)),
        ],
        out_specs=[out_spec, out_spec, out_spec],
        out_shape=out_shape,
    )(flat, sim_matrix, g2)
    return rw, lg, am


# speedup vs baseline: 2.9778x; 1.0016x over previous
import functools

import jax
import jax.numpy as jnp
from jax.experimental import pallas as pl
from jax.experimental.pallas import tpu as pltpu

_NE = 16
_MIN_EXPERTS = 2


def _gating_body(x_ref, w_ref, g_ref, rw_ref, lg_ref, am_ref):
    x = x_ref[...]
    w = w_ref[...]
    g = g_ref[...]

    wn = w * jax.lax.rsqrt(jnp.maximum(jnp.sum(w * w, axis=0, keepdims=True),
                                       1e-24))
    rn = jnp.sqrt(jnp.sum(x * x, axis=1, keepdims=True))
    xn = x / jnp.maximum(rn, 1e-12)
    xw = jnp.dot(xn.astype(jnp.bfloat16), wn.astype(jnp.bfloat16),
                 preferred_element_type=jnp.float32)
    logits = xw - jax.nn.sigmoid(g)

    gated = jnp.maximum(logits, 0.0)
    act = (logits > 0.0).astype(jnp.float32)
    inactive = jnp.sum(act, axis=1, keepdims=True) == 0.0

    col = jax.lax.broadcasted_iota(jnp.int32, logits.shape, 1)
    a1 = jnp.argmax(logits, axis=1, keepdims=True)
    masked1 = jnp.where(col == a1, -jnp.inf, logits)
    a2 = jnp.argmax(masked1, axis=1, keepdims=True)
    fb = ((col == a1) | (col == a2)).astype(jnp.float32)

    mask = jnp.where(inactive, fb, act)
    neg_inf = jnp.float32(-jnp.inf)
    ml = jnp.where(mask > 0.0, gated, neg_inf)
    mx = jnp.max(ml, axis=1, keepdims=True)
    e = jnp.exp(ml - mx)
    rw = e / jnp.sum(e, axis=1, keepdims=True)

    rw_ref[...] = rw
    lg_ref[...] = logits
    am_ref[...] = mask


@functools.partial(jax.jit, static_argnames=())
def kernel(hidden_states, sim_matrix, gates):
    b, t, c = hidden_states.shape
    n = b * t
    flat = hidden_states.reshape(n, c)
    g2 = gates.reshape(1, _NE)

    tile_m = 1024
    grid = (n // tile_m,)

    out_shape = [
        jax.ShapeDtypeStruct((n, _NE), jnp.float32),
        jax.ShapeDtypeStruct((n, _NE), jnp.float32),
        jax.ShapeDtypeStruct((n, _NE), jnp.float32),
    ]
    out_spec = pl.BlockSpec((tile_m, _NE), lambda i: (i, 0))

    rw, lg, am = pl.pallas_call(
        _gating_body,
        grid=grid,
        in_specs=[
            pl.BlockSpec((tile_m, c), lambda i: (i, 0)),
            pl.BlockSpec((c, _NE), lambda i: (0, 0)),
            pl.BlockSpec((1, _NE), lambda i: (0, 0)),
        ],
        out_specs=[out_spec, out_spec, out_spec],
        out_shape=out_shape,
        compiler_params=pltpu.CompilerParams(
            dimension_semantics=("parallel",),
        ),
    )(flat, sim_matrix, g2)
    return rw, lg, am


# speedup vs baseline: 3.0928x; 1.0386x over previous
import functools

import jax
import jax.numpy as jnp
from jax.experimental import pallas as pl
from jax.experimental.pallas import tpu as pltpu

_NE = 16
_MIN_EXPERTS = 2


def _gating_body(x_ref, w_ref, g_ref, rw_ref, lg_ref, am_ref):
    x = x_ref[...]
    w = w_ref[...]
    g = g_ref[...]

    wn = w * jax.lax.rsqrt(jnp.maximum(jnp.sum(w * w, axis=0, keepdims=True),
                                       1e-24))
    rn = jnp.sqrt(jnp.sum(x * x, axis=1, keepdims=True))
    xn = x / jnp.maximum(rn, 1e-12)
    xw = jnp.dot(xn.astype(jnp.bfloat16), wn.astype(jnp.bfloat16),
                 preferred_element_type=jnp.float32)
    logits = xw - jax.nn.sigmoid(g)

    gated = jnp.maximum(logits, 0.0)
    act = (logits > 0.0).astype(jnp.float32)
    inactive = jnp.sum(act, axis=1, keepdims=True) == 0.0

    col = jax.lax.broadcasted_iota(jnp.int32, logits.shape, 1)
    a1 = jnp.argmax(logits, axis=1, keepdims=True)
    masked1 = jnp.where(col == a1, -jnp.inf, logits)
    a2 = jnp.argmax(masked1, axis=1, keepdims=True)
    fb = ((col == a1) | (col == a2)).astype(jnp.float32)

    mask = jnp.where(inactive, fb, act)
    neg_inf = jnp.float32(-jnp.inf)
    ml = jnp.where(mask > 0.0, gated, neg_inf)
    mx = jnp.max(ml, axis=1, keepdims=True)
    e = jnp.exp(ml - mx)
    rw = e / jnp.sum(e, axis=1, keepdims=True)

    rw_ref[...] = rw
    lg_ref[...] = logits
    am_ref[...] = mask


@functools.partial(jax.jit, static_argnames=())
def kernel(hidden_states, sim_matrix, gates):
    b, t, c = hidden_states.shape
    n = b * t
    flat = hidden_states.reshape(n, c)
    g2 = gates.reshape(1, _NE)

    tile_m = 2048
    grid = (n // tile_m,)

    out_shape = [
        jax.ShapeDtypeStruct((n, _NE), jnp.float32),
        jax.ShapeDtypeStruct((n, _NE), jnp.float32),
        jax.ShapeDtypeStruct((n, _NE), jnp.float32),
    ]
    out_spec = pl.BlockSpec((tile_m, _NE), lambda i: (i, 0))

    rw, lg, am = pl.pallas_call(
        _gating_body,
        grid=grid,
        in_specs=[
            pl.BlockSpec((tile_m, c), lambda i: (i, 0)),
            pl.BlockSpec((c, _NE), lambda i: (0, 0)),
            pl.BlockSpec((1, _NE), lambda i: (0, 0)),
        ],
        out_specs=[out_spec, out_spec, out_spec],
        out_shape=out_shape,
        compiler_params=pltpu.CompilerParams(
            dimension_semantics=("parallel",),
        ),
    )(flat, sim_matrix, g2)
    return rw, lg, am
